# baseline (speedup 1.0000x reference)
import functools

import jax
import jax.numpy as jnp
from jax import lax
from jax.experimental import pallas as pl
from jax.experimental.pallas import tpu as pltpu
from jax.experimental.pallas import tpu_sc as plsc

B = 4096
F = 26
ROWS = 100001
D = 64

NC = 2
NS = 16
NW = NC * NS

TILE_B = 128
TILES_PER_F = B // TILE_B
NTILES = F * TILES_PER_F
TPW = NTILES // NW

_mesh = plsc.VectorSubcoreMesh(core_axis_name="c", subcore_axis_name="s")


@functools.partial(
    pl.kernel,
    mesh=_mesh,
    out_type=jax.ShapeDtypeStruct((B, F, D), jnp.float32),
    scratch_types=[
        pltpu.VMEM((TILE_B,), jnp.int32),
        pltpu.VMEM((TILE_B, D), jnp.float32),
        pltpu.VMEM((D,), jnp.float32),
        pltpu.SemaphoreType.DMA,
    ],
    compiler_params=pltpu.CompilerParams(use_tc_tiling_on_sc=False),
)
def _gather_bias(tab_hbm, idx_hbm, bias_hbm, out_hbm, idx_v, rows_v, bias_v, sem):
    wid = lax.axis_index("s") * NC + lax.axis_index("c")

    def tile_body(j, _):
        t = wid * TPW + j
        f = t // TILES_PER_F
        b0 = (t % TILES_PER_F) * TILE_B
        pltpu.sync_copy(idx_hbm.at[pl.ds(f * B + b0, TILE_B)], idx_v)
        pltpu.async_copy(tab_hbm.at[f].at[idx_v], rows_v, sem).wait()
        pltpu.sync_copy(bias_hbm.at[f], bias_v)
        b_regs = [bias_v[pl.ds(k * 16, 16)] for k in range(4)]

        def row_body(r, _):
            for k in range(4):
                sl = pl.ds(k * 16, 16)
                rows_v[r, sl] = rows_v[r, sl] + b_regs[k]
            return 0

        lax.fori_loop(0, TILE_B, row_body, 0)
        pltpu.sync_copy(rows_v, out_hbm.at[pl.ds(b0, TILE_B), f])
        return 0

    lax.fori_loop(0, TPW, tile_body, 0)


def kernel(cat_features, tables, bias):
    idx = cat_features.T.astype(jnp.int32).reshape(-1)
    out = _gather_bias(tables, idx, bias)
    return out.reshape(B, F * D)


# speedup vs baseline: 7.0773x; 7.0773x over previous
import functools

import jax
import jax.numpy as jnp
from jax import lax
from jax.experimental import pallas as pl
from jax.experimental.pallas import tpu as pltpu
from jax.experimental.pallas import tpu_sc as plsc

B = 4096
F = 26
ROWS = 100001
D = 64

NC = 2
NS = 16
NW = NC * NS

P = F // 2
TILE_B = 128
TILES_PER_P = B // TILE_B
NTILES = P * TILES_PER_P
TPW = NTILES // NW
CHUNK = 32

_mesh = plsc.VectorSubcoreMesh(core_axis_name="c", subcore_axis_name="s")


def _lane(v, l):
    return jnp.squeeze(lax.slice(v, (l,), (l + 1,)))


@functools.partial(
    pl.kernel,
    mesh=_mesh,
    out_type=jax.ShapeDtypeStruct((B, F * D), jnp.float32),
    scratch_types=[
        pltpu.VMEM((TILE_B,), jnp.int32),
        pltpu.VMEM((2, CHUNK, 8, D), jnp.float32),
        pltpu.VMEM((TILE_B, 2 * D), jnp.float32),
        pltpu.VMEM((16, 2 * D), jnp.float32),
        pltpu.SemaphoreType.DMA((2,)),
    ],
    compiler_params=pltpu.CompilerParams(use_tc_tiling_on_sc=True),
)
def _gather_bias(tab_hbm, idx_hbm, bias_hbm, out_hbm,
                 idx_v, stage_v, tile_v, bias_v, sem):
    wid = lax.axis_index("s") * NC + lax.axis_index("c")
    pltpu.sync_copy(bias_hbm, bias_v)

    def tile_body(j, _):
        t = wid * TPW + j
        p = t // TILES_PER_P
        b0 = pl.multiple_of((t % TILES_PER_P) * TILE_B, TILE_B)

        for half in range(2):
            f = 2 * p + half
            pltpu.sync_copy(
                idx_hbm.at[pl.ds(pl.multiple_of(f * B + b0, TILE_B), TILE_B)],
                idx_v)

            b_regs = [bias_v[p, pl.ds(half * D + k * 16, 16)] for k in range(4)]

            def fire(c):
                s = c % 2
                for g in range(CHUNK // 16):
                    v = idx_v[pl.ds(c * CHUNK + g * 16, 16)]
                    for l in range(16):
                        i = _lane(v, l)
                        a = pl.multiple_of((i // 8) * 8, 8)
                        pltpu.async_copy(tab_hbm.at[f, pl.ds(a, 8)],
                                         stage_v.at[s, g * 16 + l], sem.at[s])

            def process(c):
                s = c % 2

                def drain(rr, _):
                    pltpu.make_async_copy(
                        tab_hbm.at[f, pl.ds(0, 8)], stage_v.at[s, rr],
                        sem.at[s]).wait()
                    return 0

                lax.fori_loop(0, CHUNK, drain, 0)
                for g in range(CHUNK // 16):
                    v = idx_v[pl.ds(c * CHUNK + g * 16, 16)]
                    sub_v = lax.rem(v, 8)
                    for l in range(16):
                        sub = _lane(sub_v, l)
                        rr = g * 16 + l
                        for k in range(4):
                            tile_v[c * CHUNK + rr,
                                   pl.ds(half * D + k * 16, 16)] = (
                                stage_v[s, rr, sub, pl.ds(k * 16, 16)]
                                + b_regs[k])

            fire(0)

            def pipe(c, _):
                fire(c + 1)
                process(c)
                return 0

            lax.fori_loop(0, TILE_B // CHUNK - 1, pipe, 0)
            process(TILE_B // CHUNK - 1)

        pltpu.sync_copy(
            tile_v,
            out_hbm.at[pl.ds(b0, TILE_B),
                       pl.ds(pl.multiple_of(p * 2 * D, 2 * D), 2 * D)])
        return 0

    lax.fori_loop(0, TPW, tile_body, 0)


def kernel(cat_features, tables, bias):
    idx = cat_features.T.astype(jnp.int32).reshape(-1)
    bias_pairs = jnp.pad(bias.reshape(P, 2 * D), ((0, 3), (0, 0)))
    return _gather_bias(tables, idx, bias_pairs)


# speedup vs baseline: 12.6444x; 1.7866x over previous
import functools

import jax
import jax.numpy as jnp
from jax import lax
from jax.experimental import pallas as pl
from jax.experimental.pallas import tpu as pltpu
from jax.experimental.pallas import tpu_sc as plsc

B = 4096
F = 26
ROWS = 100001
D = 64

NC = 2
NS = 16
NW = NC * NS

P = F // 2
TILE_B = 128
TILES_PER_P = B // TILE_B
NTILES = P * TILES_PER_P
TPW = NTILES // NW

_mesh = plsc.VectorSubcoreMesh(core_axis_name="c", subcore_axis_name="s")


@functools.partial(
    pl.kernel,
    mesh=_mesh,
    out_type=jax.ShapeDtypeStruct((B, F * D), jnp.float32),
    scratch_types=[
        pltpu.VMEM((2, TILE_B), jnp.int32),
        pltpu.VMEM((2, TILE_B, 2 * D), jnp.float32),
        pltpu.VMEM((TILE_B, 2 * D), jnp.float32),
        pltpu.VMEM((16, 2 * D), jnp.float32),
        pltpu.SemaphoreType.DMA((2,)),
    ],
    compiler_params=pltpu.CompilerParams(use_tc_tiling_on_sc=True),
)
def _gather_bias(tab_hbm, idx_hbm, bias_hbm, out_hbm,
                 idx_v, stage_v, tile_v, bias_v, sem):
    wid = lax.axis_index("s") * NC + lax.axis_index("c")
    pltpu.sync_copy(bias_hbm, bias_v)

    def tile_body(j, _):
        t = wid * TPW + j
        p = t // TILES_PER_P
        b0 = pl.multiple_of((t % TILES_PER_P) * TILE_B, TILE_B)

        def fire(half):
            f = 2 * p + half
            pltpu.sync_copy(
                idx_hbm.at[pl.ds(pl.multiple_of(f * B + b0, TILE_B), TILE_B)],
                idx_v.at[half])
            pltpu.async_copy(tab_hbm.at[p].at[idx_v.at[half]],
                             stage_v.at[half], sem.at[half])

        def process(half):
            pltpu.make_async_copy(
                tab_hbm.at[p, pl.ds(0, TILE_B)], stage_v.at[half],
                sem.at[half]).wait()
            b_regs = [bias_v[p, pl.ds(half * D + k * 16, 16)]
                      for k in range(4)]

            def row_body(r, _):
                for k in range(4):
                    tile_v[r, pl.ds(half * D + k * 16, 16)] = (
                        stage_v[half, r, pl.ds(half * D + k * 16, 16)]
                        + b_regs[k])
                return 0

            lax.fori_loop(0, TILE_B, row_body, 0)

        fire(0)
        fire(1)
        process(0)
        process(1)
        pltpu.sync_copy(
            tile_v,
            out_hbm.at[pl.ds(b0, TILE_B),
                       pl.ds(pl.multiple_of(p * 2 * D, 2 * D), 2 * D)])
        return 0

    lax.fori_loop(0, TPW, tile_body, 0)


def kernel(cat_features, tables, bias):
    idx = cat_features.T.astype(jnp.int32).reshape(-1)
    tab_p = tables.reshape(P, 2, ROWS, D).transpose(0, 2, 1, 3).reshape(
        P, ROWS, 2 * D)
    bias_pairs = jnp.pad(bias.reshape(P, 2 * D), ((0, 3), (0, 0)))
    return _gather_bias(tab_p, idx, bias_pairs)
